# baseline (speedup 1.0000x reference)
import functools

import jax
import jax.numpy as jnp
from jax import lax
from jax.experimental import pallas as pl
from jax.experimental.pallas import tpu as pltpu
from jax.experimental.pallas import tpu_sc as plsc


def _sc_gather_sum(B, HIST, D):
    info = plsc.get_sparse_core_info()
    nc, ns = info.num_cores, info.num_subcores
    nw = nc * ns
    assert B % nw == 0
    bpw = B // nw

    n_vec = D // 16

    G = 2
    NBUF = 2
    GH = G * HIST
    NG = bpw // G
    U = 8
    assert bpw % (G * NBUF) == 0 and HIST % U == 0 and GH % 8 == 0
    chunks = [(rr, o, min(128, HIST - o))
              for rr in range(G) for o in range(0, HIST, 128)]
    assert all(o % 8 == 0 for _, o, _ in chunks)

    mesh = plsc.VectorSubcoreMesh(core_axis_name="c", subcore_axis_name="s")

    @functools.partial(
        pl.kernel,
        mesh=mesh,
        compiler_params=pltpu.CompilerParams(use_tc_tiling_on_sc=False),
        out_type=jax.ShapeDtypeStruct((B, D), jnp.float32),
        scratch_types=[
            pltpu.VMEM((bpw, HIST), jnp.int32),
            pltpu.VMEM((NBUF, GH, D), jnp.float32),
            pltpu.VMEM((bpw, D), jnp.float32),
        ] + [pltpu.SemaphoreType.DMA] * NBUF,
    )
    def sc_sum(idx_hbm, table_hbm, out_hbm, idx_v, rows_v, stage_v, *sems):
        wid = lax.axis_index("s") * nc + lax.axis_index("c")
        base = wid * bpw
        pltpu.sync_copy(idx_hbm.at[pl.ds(base, bpw)], idx_v)

        def _copies(g, buf):
            return [pltpu.make_async_copy(
                        table_hbm.at[idx_v.at[g * G + rr, pl.ds(o, l)]],
                        rows_v.at[buf, pl.ds(rr * HIST + o, l)],
                        sems[buf])
                    for rr, o, l in chunks]

        def issue(g, buf):
            for c in _copies(g, buf):
                c.start()

        def drain(g, buf):
            for c in _copies(g, buf):
                c.wait()

        zero = jnp.zeros((16,), jnp.float32)

        def accum(g, buf):
            for rr in range(G):
                def body(jj, accs, _rr=rr):
                    j0 = _rr * HIST + jj * U
                    for u in range(U):
                        accs = tuple(
                            accs[k] + rows_v[buf, j0 + u, pl.ds(16 * k, 16)]
                            for k in range(n_vec))
                    return accs

                accs = lax.fori_loop(0, HIST // U, body, (zero,) * n_vec)
                r_out = g * G + rr
                for k in range(n_vec):
                    stage_v[r_out, pl.ds(16 * k, 16)] = accs[k]

        issue(0, 0)

        def outer(i, carry):
            g0 = i * NBUF
            for b in range(NBUF):
                cur = g0 + b
                nxt = cur + 1

                @pl.when(nxt < NG)
                def _(nxt=nxt, b=b):
                    issue(nxt, (b + 1) % NBUF)

                drain(cur, b)
                accum(cur, b)
            return carry

        lax.fori_loop(0, NG // NBUF, outer, 0)
        pltpu.sync_copy(stage_v, out_hbm.at[pl.ds(base, bpw)])

    return sc_sum


def _tc_linear(sums, W, b2, scale):
    B, D = sums.shape
    OUT = W.shape[1]
    blk = 512 if B % 512 == 0 else B

    def body(s_ref, w_ref, b_ref, o_ref):
        o_ref[...] = jnp.dot(s_ref[...] * scale, w_ref[...],
                             preferred_element_type=jnp.float32) + b_ref[...]

    return pl.pallas_call(
        body,
        grid=(B // blk,),
        in_specs=[
            pl.BlockSpec((blk, D), lambda i: (i, 0)),
            pl.BlockSpec((D, OUT), lambda i: (0, 0)),
            pl.BlockSpec((1, OUT), lambda i: (0, 0)),
        ],
        out_specs=pl.BlockSpec((blk, OUT), lambda i: (i, 0)),
        out_shape=jax.ShapeDtypeStruct((B, OUT), jnp.float32),
    )(sums, W, b2)


def kernel(word_indices, embedding, W, b):
    B, HIST = word_indices.shape
    D = embedding.shape[1]
    sums = _sc_gather_sum(B, HIST, D)(word_indices.astype(jnp.int32), embedding)
    return _tc_linear(sums, W, b.reshape(1, -1), 1.0 / HIST)


# speedup vs baseline: 1.0027x; 1.0027x over previous
import functools

import jax
import jax.numpy as jnp
from jax import lax
from jax.experimental import pallas as pl
from jax.experimental.pallas import tpu as pltpu
from jax.experimental.pallas import tpu_sc as plsc


def _sc_gather_sum(B, HIST, D):
    info = plsc.get_sparse_core_info()
    nc, ns = info.num_cores, info.num_subcores
    nw = nc * ns
    assert B % nw == 0
    bpw = B // nw

    n_vec = D // 16

    G = 2
    NBUF = 2
    GH = G * HIST
    NG = bpw // G
    U = 8
    assert bpw % (G * NBUF) == 0 and HIST % U == 0 and GH % 8 == 0
    chunks = [(o, min(128, GH - o)) for o in range(0, GH, 128)]
    assert all(o % 8 == 0 for o, _ in chunks)
    NR = B * HIST // 128
    rpw = bpw * HIST // 128
    assert bpw * HIST % 128 == 0

    mesh = plsc.VectorSubcoreMesh(core_axis_name="c", subcore_axis_name="s")

    @functools.partial(
        pl.kernel,
        mesh=mesh,
        compiler_params=pltpu.CompilerParams(use_tc_tiling_on_sc=False),
        out_type=jax.ShapeDtypeStruct((B, D), jnp.float32),
        scratch_types=[
            pltpu.VMEM((rpw, 128), jnp.int32),
            pltpu.VMEM((bpw * HIST,), jnp.int32),
            pltpu.VMEM((NBUF, GH, D), jnp.float32),
            pltpu.VMEM((bpw, D), jnp.float32),
        ] + [pltpu.SemaphoreType.DMA] * NBUF,
    )
    def sc_sum(idx_hbm, table_hbm, out_hbm, idx2_v, idx_v, rows_v, stage_v,
               *sems):
        wid = lax.axis_index("s") * nc + lax.axis_index("c")
        base = wid * bpw
        pltpu.sync_copy(idx_hbm.at[pl.ds(wid * rpw, rpw)], idx2_v)

        def flat_body(j, carry):
            for k in range(128 // 16):
                idx_v[pl.ds(j * 128 + 16 * k, 16)] = idx2_v[j, pl.ds(16 * k, 16)]
            return carry

        lax.fori_loop(0, rpw, flat_body, 0)

        def _copies(g, buf):
            off = pl.multiple_of(g * GH, 8)
            return [pltpu.make_async_copy(
                        table_hbm.at[idx_v.at[pl.ds(off + o, l)]],
                        rows_v.at[buf, pl.ds(o, l)],
                        sems[buf])
                    for o, l in chunks]

        def issue(g, buf):
            for c in _copies(g, buf):
                c.start()

        def drain(g, buf):
            for c in _copies(g, buf):
                c.wait()

        zero = jnp.zeros((16,), jnp.float32)

        def accum(g, buf):
            for rr in range(G):
                def body(jj, accs, _rr=rr):
                    j0 = _rr * HIST + jj * U
                    for u in range(U):
                        accs = tuple(
                            accs[k] + rows_v[buf, j0 + u, pl.ds(16 * k, 16)]
                            for k in range(n_vec))
                    return accs

                accs = lax.fori_loop(0, HIST // U, body, (zero,) * n_vec)
                r_out = g * G + rr
                for k in range(n_vec):
                    stage_v[r_out, pl.ds(16 * k, 16)] = accs[k]

        issue(0, 0)

        def outer(i, carry):
            g0 = i * NBUF
            for b in range(NBUF):
                cur = g0 + b
                nxt = cur + 1

                @pl.when(nxt < NG)
                def _(nxt=nxt, b=b):
                    issue(nxt, (b + 1) % NBUF)

                drain(cur, b)
                accum(cur, b)
            return carry

        lax.fori_loop(0, NG // NBUF, outer, 0)
        pltpu.sync_copy(stage_v, out_hbm.at[pl.ds(base, bpw)])

    return sc_sum


def _tc_linear(sums, W, b2, scale):
    B, D = sums.shape
    OUT = W.shape[1]
    blk = 512 if B % 512 == 0 else B

    def body(s_ref, w_ref, b_ref, o_ref):
        o_ref[...] = jnp.dot(s_ref[...] * scale, w_ref[...],
                             preferred_element_type=jnp.float32) + b_ref[...]

    return pl.pallas_call(
        body,
        grid=(B // blk,),
        in_specs=[
            pl.BlockSpec((blk, D), lambda i: (i, 0)),
            pl.BlockSpec((D, OUT), lambda i: (0, 0)),
            pl.BlockSpec((1, OUT), lambda i: (0, 0)),
        ],
        out_specs=pl.BlockSpec((blk, OUT), lambda i: (i, 0)),
        out_shape=jax.ShapeDtypeStruct((B, OUT), jnp.float32),
    )(sums, W, b2)


def kernel(word_indices, embedding, W, b):
    B, HIST = word_indices.shape
    D = embedding.shape[1]
    idx_rows = word_indices.astype(jnp.int32).reshape(B * HIST // 128, 128)
    sums = _sc_gather_sum(B, HIST, D)(idx_rows, embedding)
    return _tc_linear(sums, W, b.reshape(1, -1), 1.0 / HIST)
